# Initial kernel scaffold; baseline (speedup 1.0000x reference)
import jax
import jax.numpy as jnp
from jax.experimental import pallas as pl


def kernel(x, W1, b1, W2, b2):
    raise NotImplementedError("write your pallas kernel here")


# speedup vs baseline: 21.5745x; 21.5745x over previous
import jax
import jax.numpy as jnp
from jax.experimental import pallas as pl


def _fused_gcn_body(x_ref, w1_ref, b1_ref, w2_ref, b2_ref, o_ref):
    h = jnp.dot(x_ref[...], w1_ref[...], preferred_element_type=jnp.float32)
    h = jnp.maximum(h + b1_ref[...], 0.0)
    o = jnp.dot(h, w2_ref[...], preferred_element_type=jnp.float32)
    o_ref[...] = o + b2_ref[...]


def kernel(x, W1, b1, W2, b2):
    B, N, d_in = x.shape
    d_hid = W1.shape[1]
    d_out = W2.shape[1]
    x2 = x.reshape(B * N, d_in)
    rows = B * N
    BN = 2000
    grid = (pl.cdiv(rows, BN),)

    out = pl.pallas_call(
        _fused_gcn_body,
        grid=grid,
        in_specs=[
            pl.BlockSpec((BN, d_in), lambda i: (i, 0)),
            pl.BlockSpec((d_in, d_hid), lambda i: (0, 0)),
            pl.BlockSpec((1, d_hid), lambda i: (0, 0)),
            pl.BlockSpec((d_hid, d_out), lambda i: (0, 0)),
            pl.BlockSpec((1, d_out), lambda i: (0, 0)),
        ],
        out_specs=pl.BlockSpec((BN, d_out), lambda i: (i, 0)),
        out_shape=jax.ShapeDtypeStruct((rows, d_out), jnp.float32),
    )(x2, W1, b1.reshape(1, d_hid), W2, b2.reshape(1, d_out))
    return out.reshape(B, N, d_out)


# speedup vs baseline: 28.1434x; 1.3045x over previous
import jax
import jax.numpy as jnp
from jax.experimental import pallas as pl


def _fused_gcn_body(x_ref, w1_ref, b1_ref, w2_ref, b2_ref, o_ref):
    h = jnp.dot(x_ref[...], w1_ref[...], preferred_element_type=jnp.float32)
    h = jnp.maximum(h + b1_ref[...], 0.0)
    o = jnp.dot(h, w2_ref[...], preferred_element_type=jnp.float32)
    o_ref[...] = o + b2_ref[...]


def kernel(x, W1, b1, W2, b2):
    B, N, d_in = x.shape
    d_hid = W1.shape[1]
    d_out = W2.shape[1]
    x2 = x.reshape(B * N, d_in)
    rows = B * N
    BN = 4000
    grid = (pl.cdiv(rows, BN),)

    out = pl.pallas_call(
        _fused_gcn_body,
        grid=grid,
        in_specs=[
            pl.BlockSpec((BN, d_in), lambda i: (i, 0)),
            pl.BlockSpec((d_in, d_hid), lambda i: (0, 0)),
            pl.BlockSpec((1, d_hid), lambda i: (0, 0)),
            pl.BlockSpec((d_hid, d_out), lambda i: (0, 0)),
            pl.BlockSpec((1, d_out), lambda i: (0, 0)),
        ],
        out_specs=pl.BlockSpec((BN, d_out), lambda i: (i, 0)),
        out_shape=jax.ShapeDtypeStruct((rows, d_out), jnp.float32),
    )(x2, W1, b1.reshape(1, d_hid), W2, b2.reshape(1, d_out))
    return out.reshape(B, N, d_out)


# speedup vs baseline: 31.0054x; 1.1017x over previous
import jax
import jax.numpy as jnp
from jax.experimental import pallas as pl


def _fused_gcn_body(x_ref, w1_ref, b1_ref, w2_ref, b2_ref, o_ref):
    h = jnp.dot(x_ref[...], w1_ref[...], preferred_element_type=jnp.float32)
    h = jnp.maximum(h + b1_ref[...], 0.0)
    o = jnp.dot(h, w2_ref[...], preferred_element_type=jnp.float32)
    o_ref[...] = o + b2_ref[...]


def kernel(x, W1, b1, W2, b2):
    B, N, d_in = x.shape
    d_hid = W1.shape[1]
    d_out = W2.shape[1]
    x2 = x.reshape(B * N, d_in)
    rows = B * N
    BN = 10000
    grid = (pl.cdiv(rows, BN),)

    out = pl.pallas_call(
        _fused_gcn_body,
        grid=grid,
        in_specs=[
            pl.BlockSpec((BN, d_in), lambda i: (i, 0)),
            pl.BlockSpec((d_in, d_hid), lambda i: (0, 0)),
            pl.BlockSpec((1, d_hid), lambda i: (0, 0)),
            pl.BlockSpec((d_hid, d_out), lambda i: (0, 0)),
            pl.BlockSpec((1, d_out), lambda i: (0, 0)),
        ],
        out_specs=pl.BlockSpec((BN, d_out), lambda i: (i, 0)),
        out_shape=jax.ShapeDtypeStruct((rows, d_out), jnp.float32),
    )(x2, W1, b1.reshape(1, d_hid), W2, b2.reshape(1, d_out))
    return out.reshape(B, N, d_out)
